# baseline (speedup 1.0000x reference)
import functools

import jax
import jax.numpy as jnp
from jax import lax
from jax.experimental import pallas as pl
from jax.experimental.pallas import tpu as pltpu
from jax.experimental.pallas import tpu_sc as plsc

VOCAB = 100000
DIM = 300
BATCH = 4096
BAG = 50

NUM_CORES = 2
NUM_SUBCORES = 16
NW = NUM_CORES * NUM_SUBCORES
BPW = BATCH // NW
LANES = 16
DIM_PAD = 384
SCALE = 1.0 / BAG

STARTS = [16 * i for i in range(19)]
NCHUNK = len(STARTS)

_mesh = plsc.VectorSubcoreMesh(core_axis_name="c", subcore_axis_name="s")


@functools.partial(
    pl.kernel,
    mesh=_mesh,
    out_type=jax.ShapeDtypeStruct((BATCH, DIM_PAD), jnp.float32),
    scratch_types=[
        pltpu.VMEM((BPW, BAG), jnp.int32),
        pltpu.VMEM((2, 3, BAG, 128), jnp.float32),
        pltpu.VMEM((BPW, DIM_PAD), jnp.float32),
        pltpu.SemaphoreType.DMA,
        pltpu.SemaphoreType.DMA,
    ],
)
def _bow_sc(idx_hbm, table_hbm, out_hbm, idx_v, rows_v, out_v, sem0, sem1):
    wid = lax.axis_index("s") * NUM_CORES + lax.axis_index("c")
    base = wid * BPW
    sems = (sem0, sem1)

    pltpu.sync_copy(idx_hbm.at[pl.ds(base, BPW)], idx_v)

    def srcs(g):
        idx = idx_v.at[g]
        return tuple(table_hbm.at[idx, pl.ds(j * 128, 128)] for j in range(3))

    def issue(g, buf):
        for j, src in enumerate(srcs(g)):
            pltpu.async_copy(src, rows_v.at[buf, j], sems[buf])

    def wait_buf(g, buf):
        for j, src in enumerate(srcs(g)):
            pltpu.make_async_copy(src, rows_v.at[buf, j], sems[buf]).wait()

    def reduce_bag(g, buf):
        def body(r, accs):
            return tuple(
                accs[i] + rows_v[buf, STARTS[i] // 128, r,
                                 pl.ds(STARTS[i] % 128, LANES)]
                for i in range(NCHUNK)
            )

        zero = jnp.zeros((LANES,), jnp.float32)
        accs = lax.fori_loop(0, BAG, body, (zero,) * NCHUNK)
        for i in range(NCHUNK):
            out_v[g, pl.ds(STARTS[i], LANES)] = accs[i] * SCALE

    issue(0, 0)

    def pair_body(p, carry):
        for h in range(2):
            g = p * 2 + h

            @pl.when(g + 1 < BPW)
            def _():
                issue(g + 1, 1 - h)

            wait_buf(g, h)
            reduce_bag(g, h)
        return carry

    lax.fori_loop(0, BPW // 2, pair_body, 0)
    pltpu.sync_copy(out_v, out_hbm.at[pl.ds(base, BPW)])


def kernel(indices, table):
    idx = jnp.asarray(indices, jnp.int32)
    table_pad = jnp.pad(table, ((0, 0), (0, DIM_PAD - DIM)))
    return _bow_sc(idx, table_pad)[:, :DIM]


# speedup vs baseline: 1.1463x; 1.1463x over previous
import functools

import jax
import jax.numpy as jnp
from jax import lax
from jax.experimental import pallas as pl
from jax.experimental.pallas import tpu as pltpu
from jax.experimental.pallas import tpu_sc as plsc

VOCAB = 100000
DIM = 300
BATCH = 4096
BAG = 50

NUM_CORES = 2
NUM_SUBCORES = 16
NW = NUM_CORES * NUM_SUBCORES
BPW = BATCH // NW
LANES = 16
PACKED = 256
NQ = PACKED // 32
TAIL = DIM - PACKED
NT = 3
DIM_PAD = PACKED + 3 * LANES
SCALE = 1.0 / BAG

_mesh = plsc.VectorSubcoreMesh(core_axis_name="c", subcore_axis_name="s")


@functools.partial(
    pl.kernel,
    mesh=_mesh,
    out_type=jax.ShapeDtypeStruct((BATCH, DIM_PAD), jnp.float32),
    scratch_types=[
        pltpu.VMEM((BPW, BAG), jnp.int32),
        pltpu.VMEM((2, BAG, 128), jnp.int32),
        pltpu.VMEM((2, BAG, 128), jnp.float32),
        pltpu.VMEM((BPW, DIM_PAD), jnp.float32),
        pltpu.SemaphoreType.DMA,
        pltpu.SemaphoreType.DMA,
    ],
)
def _bow_sc(idx_hbm, tw_hbm, tail_hbm, out_hbm, idx_v, roww_v, rowt_v, out_v,
            sem0, sem1):
    wid = lax.axis_index("s") * NUM_CORES + lax.axis_index("c")
    base = wid * BPW
    sems = (sem0, sem1)

    pltpu.sync_copy(idx_hbm.at[pl.ds(base, BPW)], idx_v)

    def issue(g, buf):
        idx = idx_v.at[g]
        pltpu.async_copy(tw_hbm.at[idx], roww_v.at[buf], sems[buf])
        pltpu.async_copy(tail_hbm.at[idx], rowt_v.at[buf], sems[buf])

    def wait_buf(g, buf):
        idx = idx_v.at[g]
        pltpu.make_async_copy(tw_hbm.at[idx], roww_v.at[buf], sems[buf]).wait()
        pltpu.make_async_copy(tail_hbm.at[idx], rowt_v.at[buf], sems[buf]).wait()

    hi_mask = jnp.full((LANES,), -65536, jnp.int32)

    def reduce_bag(g, buf):
        def body(r, accs):
            new = []
            for q in range(NQ):
                w = roww_v[buf, r, pl.ds(LANES * q, LANES)]
                even = lax.bitcast_convert_type(lax.shift_left(w, 16), jnp.float32)
                odd = lax.bitcast_convert_type(
                    lax.bitwise_and(w, hi_mask), jnp.float32
                )
                new.append(accs[2 * q] + even)
                new.append(accs[2 * q + 1] + odd)
            for t in range(NT):
                v = rowt_v[buf, r, pl.ds(LANES * t, LANES)]
                new.append(accs[2 * NQ + t] + v)
            return tuple(new)

        zero = jnp.zeros((LANES,), jnp.float32)
        accs = lax.fori_loop(0, BAG, body, (zero,) * (2 * NQ + NT))
        for q in range(NQ):
            out_v[g, pl.ds(32 * q, LANES)] = accs[2 * q] * SCALE
            out_v[g, pl.ds(32 * q + LANES, LANES)] = accs[2 * q + 1] * SCALE
        for t in range(NT):
            out_v[g, pl.ds(PACKED + LANES * t, LANES)] = accs[2 * NQ + t] * SCALE

    issue(0, 0)

    def pair_body(p, carry):
        for h in range(2):
            g = p * 2 + h

            @pl.when(g + 1 < BPW)
            def _():
                issue(g + 1, 1 - h)

            wait_buf(g, h)
            reduce_bag(g, h)
        return carry

    lax.fori_loop(0, BPW // 2, pair_body, 0)
    pltpu.sync_copy(out_v, out_hbm.at[pl.ds(base, BPW)])


def kernel(indices, table):
    idx = jnp.asarray(indices, jnp.int32)
    tw = lax.bitcast_convert_type(
        table[:, :PACKED].astype(jnp.bfloat16).reshape(VOCAB, 128, 2),
        jnp.int32,
    )
    tail = jnp.pad(table[:, PACKED:], ((0, 0), (0, 128 - TAIL)))
    outp = _bow_sc(idx, tw, tail)
    main = (
        outp[:, :PACKED]
        .reshape(BATCH, NQ, 2, LANES)
        .transpose(0, 1, 3, 2)
        .reshape(BATCH, PACKED)
    )
    return jnp.concatenate([main, outp[:, PACKED:DIM]], axis=1)


# speedup vs baseline: 1.9050x; 1.6619x over previous
import functools

import jax
import jax.numpy as jnp
from jax import lax
from jax.experimental import pallas as pl
from jax.experimental.pallas import tpu as pltpu
from jax.experimental.pallas import tpu_sc as plsc

VOCAB = 100000
DIM = 300
BATCH = 4096
BAG = 50

NUM_CORES = 2
NUM_SUBCORES = 16
NW = NUM_CORES * NUM_SUBCORES
BPW = BATCH // NW
LANES = 16
TILE = 128
MAIN = 2 * TILE
TAIL = DIM - MAIN
TAIL_PAD = 3 * LANES
SCALE = 1.0 / BAG

_mesh = plsc.VectorSubcoreMesh(core_axis_name="c", subcore_axis_name="s")


def _worker_base():
    wid = lax.axis_index("s") * NUM_CORES + lax.axis_index("c")
    return wid * BPW


def _run_bags(issue, wait_buf, reduce_bag):
    issue(0, 0)

    def pair_body(p, carry):
        for h in range(2):
            g = p * 2 + h

            @pl.when(g + 1 < BPW)
            def _():
                issue(g + 1, 1 - h)

            wait_buf(g, h)
            reduce_bag(g, h)
        return carry

    lax.fori_loop(0, BPW // 2, pair_body, 0)


@functools.partial(
    pl.kernel,
    mesh=_mesh,
    out_type=jax.ShapeDtypeStruct((BATCH, MAIN), jnp.float32),
    scratch_types=[
        pltpu.VMEM((BPW, BAG), jnp.int32),
        pltpu.VMEM((2, 2, BAG, TILE), jnp.float32),
        pltpu.VMEM((BPW, MAIN), jnp.float32),
        pltpu.SemaphoreType.DMA,
        pltpu.SemaphoreType.DMA,
    ],
)
def _bow_main(idx_hbm, table_hbm, out_hbm, idx_v, rows_v, out_v, sem0, sem1):
    base = _worker_base()
    sems = (sem0, sem1)
    pltpu.sync_copy(idx_hbm.at[pl.ds(base, BPW)], idx_v)

    def srcs(g):
        idx = idx_v.at[g]
        return tuple(table_hbm.at[idx, pl.ds(j * TILE, TILE)] for j in range(2))

    def issue(g, buf):
        for j, src in enumerate(srcs(g)):
            pltpu.async_copy(src, rows_v.at[buf, j], sems[buf])

    def wait_buf(g, buf):
        for j, src in enumerate(srcs(g)):
            pltpu.make_async_copy(src, rows_v.at[buf, j], sems[buf]).wait()

    def reduce_bag(g, buf):
        def body(r, accs):
            new = []
            for i in range(MAIN // LANES):
                j, col = divmod(LANES * i, TILE)
                new.append(accs[i] + rows_v[buf, j, r, pl.ds(col, LANES)])
            return tuple(new)

        zero = jnp.zeros((LANES,), jnp.float32)
        accs = lax.fori_loop(0, BAG, body, (zero,) * (MAIN // LANES))
        for i in range(MAIN // LANES):
            out_v[g, pl.ds(LANES * i, LANES)] = accs[i] * SCALE

    _run_bags(issue, wait_buf, reduce_bag)
    pltpu.sync_copy(out_v, out_hbm.at[pl.ds(base, BPW)])


@functools.partial(
    pl.kernel,
    mesh=_mesh,
    out_type=jax.ShapeDtypeStruct((BATCH, TAIL_PAD), jnp.float32),
    scratch_types=[
        pltpu.VMEM((BPW, BAG), jnp.int32),
        pltpu.VMEM((2, BAG, TILE), jnp.float32),
        pltpu.VMEM((BPW, TAIL_PAD), jnp.float32),
        pltpu.SemaphoreType.DMA,
        pltpu.SemaphoreType.DMA,
    ],
)
def _bow_tail(idx_hbm, tail_hbm, out_hbm, idx_v, rows_v, out_v, sem0, sem1):
    base = _worker_base()
    sems = (sem0, sem1)
    pltpu.sync_copy(idx_hbm.at[pl.ds(base, BPW)], idx_v)

    def issue(g, buf):
        pltpu.async_copy(tail_hbm.at[idx_v.at[g]], rows_v.at[buf], sems[buf])

    def wait_buf(g, buf):
        pltpu.make_async_copy(
            tail_hbm.at[idx_v.at[g]], rows_v.at[buf], sems[buf]
        ).wait()

    def reduce_bag(g, buf):
        def body(r, accs):
            return tuple(
                accs[i] + rows_v[buf, r, pl.ds(LANES * i, LANES)]
                for i in range(TAIL_PAD // LANES)
            )

        zero = jnp.zeros((LANES,), jnp.float32)
        accs = lax.fori_loop(0, BAG, body, (zero,) * (TAIL_PAD // LANES))
        for i in range(TAIL_PAD // LANES):
            out_v[g, pl.ds(LANES * i, LANES)] = accs[i] * SCALE

    _run_bags(issue, wait_buf, reduce_bag)
    pltpu.sync_copy(out_v, out_hbm.at[pl.ds(base, BPW)])


def kernel(indices, table):
    idx = jnp.asarray(indices, jnp.int32)
    tail = jnp.pad(table[:, MAIN:], ((0, 0), (0, TILE - TAIL)))
    out_main = _bow_main(idx, table)
    out_tail = _bow_tail(idx, tail)
    return jnp.concatenate([out_main, out_tail[:, :TAIL]], axis=1)
